# baseline (speedup 1.0000x reference)
import functools

import jax
import jax.numpy as jnp
from jax import lax
from jax.experimental import pallas as pl
from jax.experimental.pallas import tpu as pltpu
from jax.experimental.pallas import tpu_sc as plsc

B = 16384
K = 512
D = 32

NSPLIT = 4
BH = B // NSPLIT

BLK = 2048
NBH = BH // BLK

DPAD = 128
NC = 2
NS = 16
NW = NC * NS
BPW = BH // NW
IDXC = 128
NCHUNK = BPW // IDXC


def _argmin_tc(ctxt_ref, anc_ref, idx_ref):
    ctxt = ctxt_ref[...]
    anc = anc_ref[...]
    an2 = jnp.sum(anc * anc, axis=1, keepdims=True)
    dots = lax.dot_general(
        anc, ctxt, (((1,), (0,)), ((), ())),
        preferred_element_type=jnp.float32,
        precision=lax.Precision.HIGHEST)
    scores = an2 - 2.0 * dots
    idx_ref[0, 0, :] = jnp.argmin(scores, axis=0).astype(jnp.int32)


def _add_tc(ctxt_ref, *refs):
    g_refs, outt_ref = refs[:-1], refs[-1]
    i = pl.program_id(0)
    g = g_refs[-1][:, :D]
    for h in range(NSPLIT - 2, -1, -1):
        g = jnp.where(i < (h + 1) * NBH, g_refs[h][:, :D], g)
    outt_ref[...] = ctxt_ref[...] + g.T


@functools.cache
def _build_gather_sc():
    mesh = plsc.VectorSubcoreMesh(core_axis_name="c", subcore_axis_name="s",
                                  num_cores=NC)

    @functools.partial(
        pl.kernel,
        mesh=mesh,
        out_type=jax.ShapeDtypeStruct((BH, DPAD), jnp.float32),
        scratch_types=[
            pltpu.VMEM((NCHUNK, IDXC), jnp.int32),
            pltpu.VMEM((BPW, DPAD), jnp.float32),
            pltpu.SemaphoreType.DMA,
            pltpu.SemaphoreType.DMA,
            pltpu.SemaphoreType.DMA,
        ],
    )
    def _gather_sc(anc_hbm, idx_hbm, g_hbm,
                   idx_v, rows_v, gsem0, gsem1, osem):
        wid = lax.axis_index("s") * NC + lax.axis_index("c")
        base = wid * BPW
        gsems = (gsem0, gsem1)
        pltpu.sync_copy(idx_hbm.at[wid], idx_v)
        gather_cp = [pltpu.async_copy(anc_hbm.at[idx_v.at[c]],
                                      rows_v.at[pl.ds(c * IDXC, IDXC)],
                                      gsems[c])
                     for c in range(NCHUNK)]
        for cp in gather_cp:
            cp.wait()
        pltpu.sync_copy(rows_v, g_hbm.at[pl.ds(base, BPW)])

    return _gather_sc


def _argmin_half(ctxt, anchors, h):
    return pl.pallas_call(
        _argmin_tc,
        grid=(NBH,),
        in_specs=[
            pl.BlockSpec((D, BLK), lambda i, _h=h: (0, i + _h * NBH)),
            pl.BlockSpec((K, D), lambda i: (0, 0)),
        ],
        out_specs=pl.BlockSpec((1, 1, BLK), lambda i: (i, 0, 0)),
        out_shape=jax.ShapeDtypeStruct((NBH, 1, BLK), jnp.int32),
        name=f"argmin_h{h}",
    )(ctxt, anchors)


def kernel(context_vector, anchors):
    ctxt = context_vector.T
    anc_pad = jnp.pad(anchors, ((0, 0), (0, DPAD - D)))
    sc = _build_gather_sc()
    gs = []
    for h in range(NSPLIT):
        idx3 = _argmin_half(ctxt, anchors, h)
        gs.append(sc(anc_pad, idx3.reshape(NW, NCHUNK, IDXC)))
    outt = pl.pallas_call(
        _add_tc,
        grid=(B // BLK,),
        in_specs=[pl.BlockSpec((D, BLK), lambda i: (0, i))] + [
            pl.BlockSpec(
                (BLK, DPAD),
                functools.partial(
                    lambda i, _h: (jnp.clip(i - _h * NBH, 0, NBH - 1), 0),
                    _h=h))
            for h in range(NSPLIT)
        ],
        out_specs=pl.BlockSpec((D, BLK), lambda i: (0, i)),
        out_shape=jax.ShapeDtypeStruct((D, B), jnp.float32),
    )(ctxt, *gs)
    return outt.T


# speedup vs baseline: 1.0305x; 1.0305x over previous
import functools

import jax
import jax.numpy as jnp
from jax import lax
from jax.experimental import pallas as pl
from jax.experimental.pallas import tpu as pltpu
from jax.experimental.pallas import tpu_sc as plsc

B = 16384
K = 512
D = 32

NSPLIT = 4
BH = B // NSPLIT

BLK = 4096
NBH = BH // BLK

DPAD = 128
NC = 2
NS = 16
NW = NC * NS
BPW = BH // NW
IDXC = 128
NCHUNK = BPW // IDXC


def _argmin_tc(ctxt_ref, anc_ref, idx_ref):
    ctxt = ctxt_ref[...]
    anc = anc_ref[...]
    an2 = jnp.sum(anc * anc, axis=1, keepdims=True)
    dots = lax.dot_general(
        anc, ctxt, (((1,), (0,)), ((), ())),
        preferred_element_type=jnp.float32,
        precision=lax.Precision.HIGHEST)
    scores = an2 - 2.0 * dots
    idx_ref[0, 0, :] = jnp.argmin(scores, axis=0).astype(jnp.int32)


def _add_tc(ctxt_ref, *refs):
    g_refs, outt_ref = refs[:-1], refs[-1]
    i = pl.program_id(0)
    g = g_refs[-1][:, :D]
    for h in range(NSPLIT - 2, -1, -1):
        g = jnp.where(i < (h + 1) * NBH, g_refs[h][:, :D], g)
    outt_ref[...] = ctxt_ref[...] + g.T


@functools.cache
def _build_gather_sc():
    mesh = plsc.VectorSubcoreMesh(core_axis_name="c", subcore_axis_name="s",
                                  num_cores=NC)

    @functools.partial(
        pl.kernel,
        mesh=mesh,
        out_type=jax.ShapeDtypeStruct((BH, DPAD), jnp.float32),
        scratch_types=[
            pltpu.VMEM((NCHUNK, IDXC), jnp.int32),
            pltpu.VMEM((BPW, DPAD), jnp.float32),
            pltpu.SemaphoreType.DMA,
            pltpu.SemaphoreType.DMA,
            pltpu.SemaphoreType.DMA,
        ],
    )
    def _gather_sc(anc_hbm, idx_hbm, g_hbm,
                   idx_v, rows_v, gsem0, gsem1, osem):
        wid = lax.axis_index("s") * NC + lax.axis_index("c")
        base = wid * BPW
        gsems = (gsem0, gsem1)
        pltpu.sync_copy(idx_hbm.at[wid], idx_v)
        gather_cp = [pltpu.async_copy(anc_hbm.at[idx_v.at[c]],
                                      rows_v.at[pl.ds(c * IDXC, IDXC)],
                                      gsems[c])
                     for c in range(NCHUNK)]
        for cp in gather_cp:
            cp.wait()
        pltpu.sync_copy(rows_v, g_hbm.at[pl.ds(base, BPW)])

    return _gather_sc


def _argmin_half(ctxt, anchors, h):
    return pl.pallas_call(
        _argmin_tc,
        grid=(NBH,),
        in_specs=[
            pl.BlockSpec((D, BLK), lambda i, _h=h: (0, i + _h * NBH)),
            pl.BlockSpec((K, D), lambda i: (0, 0)),
        ],
        out_specs=pl.BlockSpec((1, 1, BLK), lambda i: (i, 0, 0)),
        out_shape=jax.ShapeDtypeStruct((NBH, 1, BLK), jnp.int32),
        name=f"argmin_h{h}",
    )(ctxt, anchors)


def kernel(context_vector, anchors):
    ctxt = context_vector.T
    anc_pad = jnp.pad(anchors, ((0, 0), (0, DPAD - D)))
    sc = _build_gather_sc()
    gs = []
    for h in range(NSPLIT):
        idx3 = _argmin_half(ctxt, anchors, h)
        gs.append(sc(anc_pad, idx3.reshape(NW, NCHUNK, IDXC)))
    outt = pl.pallas_call(
        _add_tc,
        grid=(B // BLK,),
        in_specs=[pl.BlockSpec((D, BLK), lambda i: (0, i))] + [
            pl.BlockSpec(
                (BLK, DPAD),
                functools.partial(
                    lambda i, _h: (jnp.clip(i - _h * NBH, 0, NBH - 1), 0),
                    _h=h))
            for h in range(NSPLIT)
        ],
        out_specs=pl.BlockSpec((D, BLK), lambda i: (0, i)),
        out_shape=jax.ShapeDtypeStruct((D, B), jnp.float32),
    )(ctxt, *gs)
    return outt.T
